# baseline (speedup 1.0000x reference)
import functools

import jax
import jax.numpy as jnp
from jax import lax
from jax.experimental import pallas as pl
from jax.experimental.pallas import tpu as pltpu
from jax.experimental.pallas import tpu_sc as plsc

N_NODES = 10000
N_EDGES = 320000
D = 128
NC = 2
NS = 16
NW = NC * NS
CPT = D // NW
FLAT = N_NODES * CPT
K = 8000
NECH = N_EDGES // K
G = K // 16


def _sc_colsum(emb_cols, pk2, vals2):
    mesh = plsc.VectorSubcoreMesh(
        core_axis_name="c", subcore_axis_name="s", num_cores=NC, num_subcores=NS
    )

    @functools.partial(
        pl.kernel,
        out_type=jax.ShapeDtypeStruct((NW, FLAT), jnp.float32),
        mesh=mesh,
        compiler_params=pltpu.CompilerParams(needs_layout_passes=False),
        scratch_types=[
            pltpu.VMEM((FLAT,), jnp.float32),
            pltpu.VMEM((FLAT,), jnp.float32),
            pltpu.VMEM((K,), jnp.int32),
            pltpu.VMEM((K,), jnp.int32),
            pltpu.VMEM((K,), jnp.float32),
            pltpu.VMEM((K,), jnp.float32),
            pltpu.SemaphoreType.DMA,
            pltpu.SemaphoreType.DMA,
            pltpu.SemaphoreType.DMA,
        ],
    )
    def body(emb_hbm, pk_hbm, vals_hbm, out_hbm,
             emb_v, acc_v, pk0, pk1, vb0, vb1, esem, ssem0, ssem1):
        cid = lax.axis_index("c")
        sid = lax.axis_index("s")
        wid = sid * NC + cid

        d_emb = pltpu.async_copy(emb_hbm.at[wid], emb_v, esem)

        zeros16 = jnp.zeros((16,), jnp.float32)

        @plsc.parallel_loop(0, FLAT // 16, unroll=8)
        def _zero(i):
            acc_v[pl.ds(i * 16, 16)] = zeros16

        pks = (pk0, pk1)
        vbs = (vb0, vb1)
        ssems = (ssem0, ssem1)

        def fire(k, b):
            pltpu.async_copy(pk_hbm.at[k], pks[b], ssems[b])
            pltpu.async_copy(vals_hbm.at[k], vbs[b], ssems[b])

        def swait(k, b):
            pltpu.make_async_copy(pk_hbm.at[k], pks[b], ssems[b]).wait()
            pltpu.make_async_copy(vals_hbm.at[k], vbs[b], ssems[b]).wait()

        fire(0, 0)
        d_emb.wait()

        def process(k, b):
            pk = pks[b]
            vb = vbs[b]

            @plsc.parallel_loop(0, G, unroll=1)
            def _groups(g):
                cv = pk[pl.ds(g * 16, 16)]
                vv = vb[pl.ds(g * 16, 16)]
                sv = lax.shift_right_logical(cv, 14)
                dv = lax.bitwise_and(cv, 16383)
                xs = [
                    plsc.load_gather(emb_v, [sv + c * N_NODES]) * vv
                    for c in range(CPT)
                ]
                for c in range(CPT):
                    plsc.addupdate_scatter(acc_v, [dv + c * N_NODES], xs[c])

        def outer(o, carry):
            for b in range(2):
                k = 2 * o + b
                swait(k, b)

                @pl.when(k + 1 < NECH)
                def _fire_next():
                    fire(k + 1, 1 - b)

                process(k, b)
            return carry

        lax.fori_loop(0, NECH // 2, outer, 0)

        pltpu.sync_copy(acc_v, out_hbm.at[wid])

    return body(emb_cols, pk2, vals2)


def kernel(users_emb, items_emb, edge_index, edge_vals):
    num_user = users_emb.shape[0]
    ucols = jnp.transpose(users_emb.reshape(num_user, NW, CPT), (1, 2, 0))
    icols = jnp.transpose(
        items_emb.reshape(N_NODES - num_user, NW, CPT), (1, 2, 0)
    )
    emb_cols = jnp.concatenate([ucols, icols], axis=2).reshape(NW, FLAT)
    dst = edge_index[0].astype(jnp.int32)
    src = edge_index[1].astype(jnp.int32)
    pk2 = (src * 16384 + dst).reshape(NECH, K)
    vals2 = edge_vals.reshape(NECH, K)
    out32 = _sc_colsum(emb_cols, pk2, vals2).reshape(NW, CPT, N_NODES)
    h_u = jnp.transpose(out32[:, :, :num_user], (2, 0, 1)).reshape(num_user, D)
    h_i = jnp.transpose(out32[:, :, num_user:], (2, 0, 1)).reshape(
        N_NODES - num_user, D
    )
    return (h_u, h_i)


# speedup vs baseline: 1.0152x; 1.0152x over previous
import functools

import jax
import jax.numpy as jnp
from jax import lax
from jax.experimental import pallas as pl
from jax.experimental.pallas import tpu as pltpu
from jax.experimental.pallas import tpu_sc as plsc

N_NODES = 10000
N_EDGES = 320000
D = 128
NC = 2
NS = 16
NW = NC * NS
CPT = D // NW
FLAT = N_NODES * CPT
K = 8000
NECH = N_EDGES // K
G = K // 16


def _sc_colsum(emb_cols, pk2, vals2):
    mesh = plsc.VectorSubcoreMesh(
        core_axis_name="c", subcore_axis_name="s", num_cores=NC, num_subcores=NS
    )

    @functools.partial(
        pl.kernel,
        out_type=jax.ShapeDtypeStruct((NW, FLAT), jnp.float32),
        mesh=mesh,
        compiler_params=pltpu.CompilerParams(needs_layout_passes=False),
        scratch_types=[
            pltpu.VMEM((FLAT,), jnp.float32),
            pltpu.VMEM((FLAT,), jnp.float32),
            pltpu.VMEM((K,), jnp.int32),
            pltpu.VMEM((K,), jnp.int32),
            pltpu.VMEM((K,), jnp.float32),
            pltpu.VMEM((K,), jnp.float32),
            pltpu.SemaphoreType.DMA,
            pltpu.SemaphoreType.DMA,
            pltpu.SemaphoreType.DMA,
        ],
    )
    def body(emb_hbm, pk_hbm, vals_hbm, out_hbm,
             emb_v, acc_v, pk0, pk1, vb0, vb1, esem, ssem0, ssem1):
        cid = lax.axis_index("c")
        sid = lax.axis_index("s")
        wid = sid * NC + cid

        d_emb = pltpu.async_copy(emb_hbm.at[wid], emb_v, esem)

        zeros16 = jnp.zeros((16,), jnp.float32)

        @plsc.parallel_loop(0, FLAT // 16, unroll=8)
        def _zero(i):
            acc_v[pl.ds(i * 16, 16)] = zeros16

        pks = (pk0, pk1)
        vbs = (vb0, vb1)
        ssems = (ssem0, ssem1)

        def fire(k, b):
            pltpu.async_copy(pk_hbm.at[k], pks[b], ssems[b])
            pltpu.async_copy(vals_hbm.at[k], vbs[b], ssems[b])

        def swait(k, b):
            pltpu.make_async_copy(pk_hbm.at[k], pks[b], ssems[b]).wait()
            pltpu.make_async_copy(vals_hbm.at[k], vbs[b], ssems[b]).wait()

        fire(0, 0)
        d_emb.wait()

        def process(k, b):
            pk = pks[b]
            vb = vbs[b]

            @plsc.parallel_loop(0, G, unroll=1)
            def _groups(g):
                cv = pk[pl.ds(g * 16, 16)]
                vv = vb[pl.ds(g * 16, 16)]
                sv = lax.shift_right_logical(cv, 14)
                dv = lax.bitwise_and(cv, 16383)
                for c in range(CPT):
                    x = plsc.load_gather(emb_v, [sv + c * N_NODES])
                    plsc.addupdate_scatter(acc_v, [dv + c * N_NODES], x * vv)

        def outer(o, carry):
            for b in range(2):
                k = 2 * o + b
                swait(k, b)

                @pl.when(k + 1 < NECH)
                def _fire_next():
                    fire(k + 1, 1 - b)

                process(k, b)
            return carry

        lax.fori_loop(0, NECH // 2, outer, 0)

        pltpu.sync_copy(acc_v, out_hbm.at[wid])

    return body(emb_cols, pk2, vals2)


def kernel(users_emb, items_emb, edge_index, edge_vals):
    num_user = users_emb.shape[0]
    ucols = jnp.transpose(users_emb.reshape(num_user, NW, CPT), (1, 2, 0))
    icols = jnp.transpose(
        items_emb.reshape(N_NODES - num_user, NW, CPT), (1, 2, 0)
    )
    emb_cols = jnp.concatenate([ucols, icols], axis=2).reshape(NW, FLAT)
    dst = edge_index[0].astype(jnp.int32)
    src = edge_index[1].astype(jnp.int32)
    pk2 = (src * 16384 + dst).reshape(NECH, K)
    vals2 = edge_vals.reshape(NECH, K)
    out32 = _sc_colsum(emb_cols, pk2, vals2).reshape(NW, CPT, N_NODES)
    h_u = jnp.transpose(out32[:, :, :num_user], (2, 0, 1)).reshape(num_user, D)
    h_i = jnp.transpose(out32[:, :, num_user:], (2, 0, 1)).reshape(
        N_NODES - num_user, D
    )
    return (h_u, h_i)
